# Initial kernel scaffold; baseline (speedup 1.0000x reference)
import jax
import jax.numpy as jnp
from jax.experimental import pallas as pl


def kernel(edges, labels, word_embeddings, W1, b1, W2, b2):
    raise NotImplementedError("write your pallas kernel here")


# speedup vs baseline: 2.2930x; 2.2930x over previous
import functools

import jax
import jax.numpy as jnp
from jax import lax
from jax.experimental import pallas as pl
from jax.experimental.pallas import tpu as pltpu
from jax.experimental.pallas import tpu_sc as plsc

N_NODES = 50000
N_EDGES = 800000
EMBED = 30
D = 32

NW = 32
GCHUNK = 128
CH = 1024
CH_ROWS = CH // GCHUNK
NCHUNK = 25
PER_W = CH * NCHUNK
PAD_E = PER_W * NW

BLK = 2000
G = N_EDGES // BLK

MUL_UNROLL = 4


def _sc_gather_mul(table, src2d, dst2d):
    mesh = plsc.VectorSubcoreMesh(core_axis_name="c", subcore_axis_name="s")

    @functools.partial(
        pl.kernel,
        mesh=mesh,
        compiler_params=pltpu.CompilerParams(use_tc_tiling_on_sc=False),
        out_type=jax.ShapeDtypeStruct((PAD_E, D), jnp.float32),
        scratch_types=[
            pltpu.VMEM((CH_ROWS, GCHUNK), jnp.int32),
            pltpu.VMEM((CH_ROWS, GCHUNK), jnp.int32),
            pltpu.VMEM((CH, D), jnp.float32),
            pltpu.VMEM((CH, D), jnp.float32),
            pltpu.SemaphoreType.DMA,
        ],
    )
    def k(table_hbm, src_hbm, dst_hbm, out_hbm, sidx, didx, srows, drows, sem):
        wid = lax.axis_index("s") * 2 + lax.axis_index("c")

        def chunk_body(i, carry):
            crow = (wid * NCHUNK + i) * CH_ROWS
            base = crow * GCHUNK
            pltpu.sync_copy(src_hbm.at[pl.ds(crow, CH_ROWS)], sidx)
            pltpu.sync_copy(dst_hbm.at[pl.ds(crow, CH_ROWS)], didx)
            cps = []
            for j in range(CH_ROWS):
                cps.append(pltpu.async_copy(
                    table_hbm.at[sidx.at[j]],
                    srows.at[pl.ds(j * GCHUNK, GCHUNK)], sem))
                cps.append(pltpu.async_copy(
                    table_hbm.at[didx.at[j]],
                    drows.at[pl.ds(j * GCHUNK, GCHUNK)], sem))
            for cp in cps:
                cp.wait()

            def mul_body(m, c2):
                for u in range(MUL_UNROLL):
                    e = m * MUL_UNROLL + u
                    for half in range(2):
                        sl = (e, pl.ds(half * 16, 16))
                        srows[sl] = srows[sl] * drows[sl]
                return c2

            lax.fori_loop(0, CH // MUL_UNROLL, mul_body, 0, unroll=False)
            pltpu.sync_copy(srows, out_hbm.at[pl.ds(base, CH)])
            return carry

        lax.fori_loop(0, NCHUNK, chunk_body, 0, unroll=False)

    return k(table, src2d, dst2d)


def _tc_mlp_loss(x, labf3, w1p, b1row, wrow, carr):
    def body(x_ref, lab_ref, w1_ref, b1_ref, w_ref, c_ref, acc_ref):
        i = pl.program_id(0)
        xb = x_ref[...]
        h = jnp.dot(xb, w1_ref[...], preferred_element_type=jnp.float32)
        h = jnp.maximum(h + b1_ref[...], 0.0)
        d = jnp.sum(h * w_ref[...], axis=1, keepdims=True) + c_ref[...]
        t = 1.0 / (1.0 + jnp.exp(-d))
        y = jnp.log(jnp.exp(t) + jnp.exp(1.0 - t))
        lf = lab_ref[0]
        s = t + lf * (1.0 - 2.0 * t)
        part = jnp.sum(y - s).reshape(1, 1)

        @pl.when(i == 0)
        def _():
            acc_ref[...] = jnp.zeros((1, 1), jnp.float32)

        acc_ref[...] += part

    return pl.pallas_call(
        body,
        grid=(G,),
        in_specs=[
            pl.BlockSpec((BLK, D), lambda i: (i, 0)),
            pl.BlockSpec((1, BLK, 1), lambda i: (i, 0, 0)),
            pl.BlockSpec((D, D), lambda i: (0, 0)),
            pl.BlockSpec((1, D), lambda i: (0, 0)),
            pl.BlockSpec((1, D), lambda i: (0, 0)),
            pl.BlockSpec((1, 1), lambda i: (0, 0)),
        ],
        out_specs=pl.BlockSpec((1, 1), lambda i: (0, 0)),
        out_shape=jax.ShapeDtypeStruct((1, 1), jnp.float32),
    )(x, labf3, w1p, b1row, wrow, carr)


def kernel(edges, labels, word_embeddings, W1, b1, W2, b2):
    src = edges[:, 0].astype(jnp.int32)
    dst = edges[:, 1].astype(jnp.int32)
    pad = PAD_E - N_EDGES
    src2d = jnp.pad(src, (0, pad)).reshape(PAD_E // GCHUNK, GCHUNK)
    dst2d = jnp.pad(dst, (0, pad)).reshape(PAD_E // GCHUNK, GCHUNK)
    table = jnp.pad(word_embeddings.astype(jnp.float32), ((0, 0), (0, D - EMBED)))

    w1p = jnp.pad(W1.astype(jnp.float32), ((0, D - EMBED), (0, D - EMBED)))
    b1row = jnp.pad(b1.astype(jnp.float32), (0, D - EMBED)).reshape(1, D)
    wrow = jnp.pad((W2[:, 0] - W2[:, 1]).astype(jnp.float32),
                   (0, D - EMBED)).reshape(1, D)
    carr = (b2[0] - b2[1]).astype(jnp.float32).reshape(1, 1)
    labf3 = labels.astype(jnp.float32).reshape(G, BLK, 1)

    x = _sc_gather_mul(table, src2d, dst2d)

    acc = _tc_mlp_loss(x, labf3, w1p, b1row, wrow, carr)
    return (acc[0, 0] / jnp.float32(N_EDGES)).astype(jnp.float32)


# speedup vs baseline: 4.9952x; 2.1784x over previous
import functools

import jax
import jax.numpy as jnp
from jax import lax
from jax.experimental import pallas as pl
from jax.experimental.pallas import tpu as pltpu
from jax.experimental.pallas import tpu_sc as plsc

N_NODES = 50000
N_EDGES = 800000
EMBED = 30
D = 32

NW = 32
GCHUNK = 128
CH = 512
CH_ROWS = CH // GCHUNK
NCHUNK = 50
PER_W = CH * NCHUNK
PAD_E = PER_W * NW
MUL_UNROLL = 4

EPR = 4
ROWL = EPR * D
BLK_E = 8000
RB = BLK_E // EPR
G = N_EDGES // BLK_E


def _sc_gather_mul(table, src2d, dst2d):
    mesh = plsc.VectorSubcoreMesh(core_axis_name="c", subcore_axis_name="s")

    @functools.partial(
        pl.kernel,
        mesh=mesh,
        compiler_params=pltpu.CompilerParams(use_tc_tiling_on_sc=False),
        out_type=jax.ShapeDtypeStruct((PAD_E, D), jnp.float32),
        scratch_types=[
            pltpu.VMEM((2, CH_ROWS, GCHUNK), jnp.int32),
            pltpu.VMEM((2, CH_ROWS, GCHUNK), jnp.int32),
            pltpu.VMEM((CH, D), jnp.float32),
            pltpu.VMEM((CH, D), jnp.float32),
            pltpu.VMEM((CH, D), jnp.float32),
            pltpu.VMEM((CH, D), jnp.float32),
            pltpu.SemaphoreType.DMA,
            pltpu.SemaphoreType.DMA,
            pltpu.SemaphoreType.DMA,
            pltpu.SemaphoreType.DMA,
        ],
    )
    def k(table_hbm, src_hbm, dst_hbm, out_hbm,
          sidx, didx, srows0, srows1, drows0, drows1, sg0, sg1, sw0, sw1):
        wid = lax.axis_index("s") * 2 + lax.axis_index("c")
        srows = (srows0, srows1)
        drows = (drows0, drows1)
        sg = (sg0, sg1)
        sw = (sw0, sw1)

        def issue(c, slot):
            crow = (wid * NCHUNK + c) * CH_ROWS
            pltpu.sync_copy(src_hbm.at[pl.ds(crow, CH_ROWS)], sidx.at[slot])
            pltpu.sync_copy(dst_hbm.at[pl.ds(crow, CH_ROWS)], didx.at[slot])
            for j in range(CH_ROWS):
                pltpu.async_copy(table_hbm.at[sidx.at[slot, j]],
                                 srows[slot].at[pl.ds(j * GCHUNK, GCHUNK)],
                                 sg[slot])
                pltpu.async_copy(table_hbm.at[didx.at[slot, j]],
                                 drows[slot].at[pl.ds(j * GCHUNK, GCHUNK)],
                                 sg[slot])

        def wait_gathers(slot):
            for j in range(CH_ROWS):
                pltpu.make_async_copy(
                    table_hbm.at[sidx.at[slot, j]],
                    srows[slot].at[pl.ds(j * GCHUNK, GCHUNK)], sg[slot]).wait()
                pltpu.make_async_copy(
                    table_hbm.at[didx.at[slot, j]],
                    drows[slot].at[pl.ds(j * GCHUNK, GCHUNK)], sg[slot]).wait()

        def drain_wb(slot):
            pltpu.make_async_copy(
                srows[slot], out_hbm.at[pl.ds(0, CH)], sw[slot]).wait()

        def step(c, slot):
            nc = c + 1

            @pl.when(nc < NCHUNK)
            def _():
                @pl.when(nc >= 2)
                def _():
                    drain_wb(1 - slot)

                issue(nc, 1 - slot)

            wait_gathers(slot)
            sr, dr = srows[slot], drows[slot]

            def mul_body(m, c2):
                for u in range(MUL_UNROLL):
                    e = m * MUL_UNROLL + u
                    for half in range(2):
                        sl = (e, pl.ds(half * 16, 16))
                        sr[sl] = sr[sl] * dr[sl]
                return c2

            lax.fori_loop(0, CH // MUL_UNROLL, mul_body, 0, unroll=False)
            base = (wid * NCHUNK + c) * CH
            pltpu.async_copy(sr, out_hbm.at[pl.ds(base, CH)], sw[slot])

        issue(0, 0)

        def pair_body(i, carry):
            step(2 * i, 0)
            step(2 * i + 1, 1)
            return carry

        lax.fori_loop(0, NCHUNK // 2, pair_body, 0, unroll=False)
        drain_wb(0)
        drain_wb(1)

    return k(table, src2d, dst2d)


def _tc_mlp_loss(x128, labr, w1big, b1big, wbig, selm, carr):
    def body(x_ref, lab_ref, w1_ref, b1_ref, w_ref, sel_ref, c_ref, acc_ref):
        i = pl.program_id(0)
        xb = x_ref[...]
        h = jnp.dot(xb, w1_ref[...], preferred_element_type=jnp.float32)
        h = jnp.maximum(h + b1_ref[...], 0.0)
        s = h * w_ref[...]
        d = jnp.dot(s, sel_ref[...],
                    preferred_element_type=jnp.float32) + c_ref[...]
        t = 1.0 / (1.0 + jnp.exp(-d))
        y = jnp.log(jnp.exp(t) + jnp.exp(1.0 - t))
        lf = lab_ref[0]
        sl = t + lf * (1.0 - 2.0 * t)
        part = jnp.sum(y - sl).reshape(1, 1)

        @pl.when(i == 0)
        def _():
            acc_ref[...] = jnp.zeros((1, 1), jnp.float32)

        acc_ref[...] += part

    return pl.pallas_call(
        body,
        grid=(G,),
        in_specs=[
            pl.BlockSpec((RB, ROWL), lambda i: (i, 0)),
            pl.BlockSpec((1, RB, EPR), lambda i: (i, 0, 0)),
            pl.BlockSpec((ROWL, ROWL), lambda i: (0, 0)),
            pl.BlockSpec((1, ROWL), lambda i: (0, 0)),
            pl.BlockSpec((1, ROWL), lambda i: (0, 0)),
            pl.BlockSpec((ROWL, EPR), lambda i: (0, 0)),
            pl.BlockSpec((1, 1), lambda i: (0, 0)),
        ],
        out_specs=pl.BlockSpec((1, 1), lambda i: (0, 0)),
        out_shape=jax.ShapeDtypeStruct((1, 1), jnp.float32),
    )(x128, labr, w1big, b1big, wbig, selm, carr)


def kernel(edges, labels, word_embeddings, W1, b1, W2, b2):
    src = edges[:, 0].astype(jnp.int32)
    dst = edges[:, 1].astype(jnp.int32)
    pad = PAD_E - N_EDGES
    src2d = jnp.pad(src, (0, pad)).reshape(PAD_E // GCHUNK, GCHUNK)
    dst2d = jnp.pad(dst, (0, pad)).reshape(PAD_E // GCHUNK, GCHUNK)
    table = jnp.pad(word_embeddings.astype(jnp.float32), ((0, 0), (0, D - EMBED)))

    eye4 = jnp.eye(EPR, dtype=jnp.float32)
    w1p = jnp.pad(W1.astype(jnp.float32), ((0, D - EMBED), (0, D - EMBED)))
    w1big = jnp.kron(eye4, w1p)
    b1big = jnp.tile(jnp.pad(b1.astype(jnp.float32), (0, D - EMBED)),
                     EPR).reshape(1, ROWL)
    wbig = jnp.tile(jnp.pad((W2[:, 0] - W2[:, 1]).astype(jnp.float32),
                            (0, D - EMBED)), EPR).reshape(1, ROWL)
    selm = jnp.kron(eye4, jnp.ones((D, 1), jnp.float32))
    carr = (b2[0] - b2[1]).astype(jnp.float32).reshape(1, 1)
    labr = labels.astype(jnp.float32).reshape(G, RB, EPR)

    x = _sc_gather_mul(table, src2d, dst2d)

    x128 = x.reshape(PAD_E // EPR, ROWL)
    acc = _tc_mlp_loss(x128, labr, w1big, b1big, wbig, selm, carr)
    return (acc[0, 0] / jnp.float32(N_EDGES)).astype(jnp.float32)
